# baseline (speedup 1.0000x reference)
import functools

import jax
import jax.numpy as jnp
from jax import lax
from jax.experimental import pallas as pl
from jax.experimental.pallas import tpu as pltpu
from jax.experimental.pallas import tpu_sc as plsc

BS = 4
NQ = 4096
NH = 8
NP = 4
HD = 32
ED = 256
GS = 64
NSLOT = NP * 4

_QT = 4
_QTR = NQ // _QT // 128

NC, NS = 2, 16
CH = 64


def _prep_body(q_ref, wof_ref, bof_ref, wat_ref, bat_ref, rx_ref, ry_ref,
               wh_ref, idx_ref, wt_ref):
    b = pl.program_id(0)
    qv = q_ref[0]
    so = jnp.maximum(
        jnp.dot(qv, wof_ref[...], preferred_element_type=jnp.float32)
        + bof_ref[...], 0.0)
    aw = jnp.maximum(
        jnp.dot(qv, wat_ref[...], preferred_element_type=jnp.float32)
        + bat_ref[...], 0.0)
    sx = 64.0 / wh_ref[0:1, 0:1]
    sy = 64.0 / wh_ref[0:1, 1:2]
    rx = rx_ref[0]
    ry = ry_ref[0]
    base = (b * NQ * NH).astype(jnp.float32)
    for hd in range(NH):
        a = aw[:, hd * NP:(hd + 1) * NP]
        m = jnp.max(a, axis=-1, keepdims=True)
        e = jnp.exp(a - m)
        s = jnp.sum(e, axis=-1, keepdims=True)
        for p in range(NP):
            ap = e[:, p:p + 1] / s
            o = (hd * NP + p) * 2
            ox = so[:, o:o + 1] * sx
            oy = so[:, o + 1:o + 2] * sy
            x = rx * 64.0 + (ox - 0.5)
            y = ry * 64.0 + (oy - 0.5)
            x0 = jnp.floor(x)
            fx = x - x0
            y0 = jnp.floor(y)
            fy = y - y0
            vx0 = ((x0 >= 0.0) & (x0 <= 63.0)).astype(jnp.float32)
            vx1 = ((x0 >= -1.0) & (x0 <= 62.0)).astype(jnp.float32)
            vy0 = ((y0 >= 0.0) & (y0 <= 63.0)).astype(jnp.float32)
            vy1 = ((y0 >= -1.0) & (y0 <= 62.0)).astype(jnp.float32)
            cx0 = jnp.clip(x0, 0.0, 63.0)
            cx1 = jnp.clip(x0 + 1.0, 0.0, 63.0)
            cy0 = jnp.clip(y0, 0.0, 63.0)
            cy1 = jnp.clip(y0 + 1.0, 0.0, 63.0)
            corners = (
                (cy0, cx0, vy0 * vx0 * (1.0 - fx) * (1.0 - fy)),
                (cy1, cx0, vy1 * vx0 * (1.0 - fx) * fy),
                (cy0, cx1, vy0 * vx1 * fx * (1.0 - fy)),
                (cy1, cx1, vy1 * vx1 * fx * fy),
            )
            for ci, (cy, cx, wc) in enumerate(corners):
                pos = cy * 64.0 + cx
                idxf = (pos * float(NH) + (base + float(hd)))
                idx_ref[0, hd, p * 4 + ci, 0] = idxf.astype(jnp.int32)
                wt_ref[0, hd, p * 4 + ci, 0] = ap * wc


def _prep_call(query, W_off, b_off, W_attn, b_attn, rx, ry, wh):
    grid = (BS, _QT)
    kwargs = dict(
        grid=grid,
        in_specs=[
            pl.BlockSpec((1, 1, ED), lambda b, t: (b, 0, 0)),
            pl.BlockSpec((ED, NH * NP * 2), lambda b, t: (0, 0)),
            pl.BlockSpec((1, NH * NP * 2), lambda b, t: (0, 0)),
            pl.BlockSpec((ED, NH * NP), lambda b, t: (0, 0)),
            pl.BlockSpec((1, NH * NP), lambda b, t: (0, 0)),
            pl.BlockSpec((1, _QTR, 128), lambda b, t: (b, t, 0)),
            pl.BlockSpec((1, _QTR, 128), lambda b, t: (b, t, 0)),
            pl.BlockSpec((1, 2), lambda b, t: (0, 0)),
        ],
        out_specs=[
            pl.BlockSpec((1, NH, NSLOT, 1, _QTR, 128),
                         lambda b, t: (b, 0, 0, t, 0, 0)),
            pl.BlockSpec((1, NH, NSLOT, 1, _QTR, 128),
                         lambda b, t: (b, 0, 0, t, 0, 0)),
        ],
        out_shape=[
            jax.ShapeDtypeStruct((BS, NH, NSLOT, _QT, _QTR, 128), jnp.int32),
            jax.ShapeDtypeStruct((BS, NH, NSLOT, _QT, _QTR, 128), jnp.float32),
        ],
    )
    return pl.pallas_call(_prep_body, **kwargs)(
        query, W_off, b_off, W_attn, b_attn, rx, ry, wh)


def _sc_body(table_hbm, idx_hbm, wt_hbm, out_hbm, idx_v, wt_v, rows_v, out_v,
             sem_a, sem_b, sem2):
    sems = (sem_a, sem_b)
    wid = lax.axis_index("s") * NC + lax.axis_index("c")
    nch = NQ // CH

    def stage_iw(buf, ch):
        qb = ch * CH
        pltpu.async_copy(idx_hbm.at[wid, :, pl.ds(qb, CH)], idx_v.at[buf],
                         sem2)
        pltpu.async_copy(wt_hbm.at[wid, pl.ds(qb, CH)], wt_v.at[buf], sem2)

    def drain_iw(buf):
        pltpu.make_async_copy(idx_hbm.at[0, :, pl.ds(0, CH)], idx_v.at[buf],
                              sem2).wait()
        pltpu.make_async_copy(wt_hbm.at[0, pl.ds(0, CH)], wt_v.at[buf],
                              sem2).wait()

    def fire(buf):
        for j in range(NSLOT):
            pltpu.async_copy(table_hbm.at[idx_v.at[buf, j]],
                             rows_v.at[buf, j], sems[buf])

    def drain_rows(buf):
        for j in range(NSLOT):
            pltpu.make_async_copy(table_hbm.at[idx_v.at[buf, j]],
                                  rows_v.at[buf, j], sems[buf]).wait()

    def compute(buf, ch):
        qb = ch * CH

        def group_body(g, c2):
            qo = g * 16
            for qi in range(16):
                q = qo + qi
                wv = wt_v[buf, q]
                acc0 = jnp.zeros((16,), jnp.float32)
                acc1 = jnp.zeros((16,), jnp.float32)
                for j in range(NSLOT):
                    w = wv[j]
                    acc0 = acc0 + w * rows_v[buf, j, q, pl.ds(0, 16)]
                    acc1 = acc1 + w * rows_v[buf, j, q, pl.ds(16, 16)]
                out_v[q, pl.ds(0, 16)] = acc0
                out_v[q, pl.ds(16, 16)] = acc1
            return c2

        lax.fori_loop(0, CH // 16, group_body, 0)
        pltpu.sync_copy(out_v, out_hbm.at[wid, pl.ds(qb, CH)])

    stage_iw(0, 0)
    drain_iw(0)
    fire(0)
    stage_iw(1, 1)

    def body(cc, carry):
        ch0 = cc * 2
        ch1 = ch0 + 1
        nxt0 = ch0 + 2
        nxt1 = ch1 + 2
        drain_iw(1)
        fire(1)
        drain_rows(0)
        compute(0, ch0)

        @pl.when(nxt0 < nch)
        def _():
            stage_iw(0, nxt0)
            drain_iw(0)
            fire(0)

        drain_rows(1)
        compute(1, ch1)

        @pl.when(nxt1 < nch)
        def _():
            stage_iw(1, nxt1)

        return carry

    lax.fori_loop(0, nch // 2, body, 0)


def _sc_call(table, idxs, wts):
    mesh = plsc.VectorSubcoreMesh(core_axis_name="c", subcore_axis_name="s")
    kern = functools.partial(
        pl.kernel, mesh=mesh,
        compiler_params=pltpu.CompilerParams(use_tc_tiling_on_sc=False),
        out_type=jax.ShapeDtypeStruct((BS * NH, NQ, HD), jnp.float32),
        scratch_types=[
            pltpu.VMEM((2, NSLOT, CH), jnp.int32),
            pltpu.VMEM((2, CH, NSLOT), jnp.float32),
            pltpu.VMEM((2, NSLOT, CH, HD), jnp.float32),
            pltpu.VMEM((CH, HD), jnp.float32),
            pltpu.SemaphoreType.DMA,
            pltpu.SemaphoreType.DMA,
            pltpu.SemaphoreType.DMA,
        ],
    )(_sc_body)
    return kern(table, idxs, wts)


def _post_body(o_ref, q_ref, g_ref, b_ref, y_ref):
    xs = [o_ref[0, hd] for hd in range(NH)]
    x = jnp.concatenate(xs, axis=-1)
    x = x + q_ref[0]
    mu = jnp.mean(x, axis=-1, keepdims=True)
    d = x - mu
    var = jnp.mean(d * d, axis=-1, keepdims=True)
    xn = d * lax.rsqrt(var + 1e-5)
    y_ref[0] = xn * g_ref[...] + b_ref[...]


def _post_call(out_bh, query, gamma, beta):
    T = NQ // _QT
    grid = (BS, _QT)
    return pl.pallas_call(
        _post_body,
        grid=grid,
        in_specs=[
            pl.BlockSpec((1, NH, T, HD), lambda b, t: (b, 0, t, 0)),
            pl.BlockSpec((1, 1, ED), lambda b, t: (b, 0, 0)),
            pl.BlockSpec((1, ED), lambda b, t: (0, 0)),
            pl.BlockSpec((1, ED), lambda b, t: (0, 0)),
        ],
        out_specs=pl.BlockSpec((1, T, ED), lambda b, t: (b, t, 0)),
        out_shape=jax.ShapeDtypeStruct((BS, NQ, ED), jnp.float32),
    )(out_bh, query, gamma, beta)


def kernel(query, single_feat, ref_2d, h, w, W_off, b_off, W_attn, b_attn,
           ln_gamma, ln_beta):
    rx = ref_2d[:, :, 0, 0].reshape(BS, NQ // 128, 128)
    ry = ref_2d[:, :, 0, 1].reshape(BS, NQ // 128, 128)
    wh = jnp.stack([jnp.asarray(w), jnp.asarray(h)]).astype(jnp.float32)
    wh = wh.reshape(1, 2)
    idx, wt = _prep_call(query, W_off, b_off.reshape(1, -1), W_attn,
                         b_attn.reshape(1, -1), rx, ry, wh)
    table = single_feat.reshape(BS * NQ * NH, HD)
    out_bh = _sc_call(table,
                      idx.reshape(BS * NH, NSLOT, NQ),
                      jnp.swapaxes(wt.reshape(BS * NH, NSLOT, NQ), 1, 2))
    y = _post_call(out_bh.reshape(BS, NH, NQ, HD), query,
                   ln_gamma.reshape(1, ED), ln_beta.reshape(1, ED))
    return y


# speedup vs baseline: 1.1960x; 1.1960x over previous
import functools

import jax
import jax.numpy as jnp
from jax import lax
from jax.experimental import pallas as pl
from jax.experimental.pallas import tpu as pltpu
from jax.experimental.pallas import tpu_sc as plsc

BS = 4
NQ = 4096
NH = 8
NP = 4
HD = 32
ED = 256
GS = 64
NSLOT = NP * 4

_QT = 4
_QTR = NQ // _QT // 128

NC, NS = 2, 16
CH = 64


def _prep_body(q_ref, wof_ref, bof_ref, wat_ref, bat_ref, rx_ref, ry_ref,
               wh_ref, idx_ref, wt_ref):
    b = pl.program_id(0)
    qv = q_ref[0]
    so = jnp.maximum(
        jnp.dot(qv, wof_ref[...], preferred_element_type=jnp.float32)
        + bof_ref[...], 0.0)
    aw = jnp.maximum(
        jnp.dot(qv, wat_ref[...], preferred_element_type=jnp.float32)
        + bat_ref[...], 0.0)
    sx = 64.0 / wh_ref[0:1, 0:1]
    sy = 64.0 / wh_ref[0:1, 1:2]
    rx = rx_ref[0]
    ry = ry_ref[0]
    base = (b * NQ * NH).astype(jnp.float32)
    for hd in range(NH):
        a = aw[:, hd * NP:(hd + 1) * NP]
        m = jnp.max(a, axis=-1, keepdims=True)
        e = jnp.exp(a - m)
        s = jnp.sum(e, axis=-1, keepdims=True)
        for p in range(NP):
            ap = e[:, p:p + 1] / s
            o = (hd * NP + p) * 2
            ox = so[:, o:o + 1] * sx
            oy = so[:, o + 1:o + 2] * sy
            x = rx * 64.0 + (ox - 0.5)
            y = ry * 64.0 + (oy - 0.5)
            x0 = jnp.floor(x)
            fx = x - x0
            y0 = jnp.floor(y)
            fy = y - y0
            vx0 = ((x0 >= 0.0) & (x0 <= 63.0)).astype(jnp.float32)
            vx1 = ((x0 >= -1.0) & (x0 <= 62.0)).astype(jnp.float32)
            vy0 = ((y0 >= 0.0) & (y0 <= 63.0)).astype(jnp.float32)
            vy1 = ((y0 >= -1.0) & (y0 <= 62.0)).astype(jnp.float32)
            cx0 = jnp.clip(x0, 0.0, 63.0)
            cx1 = jnp.clip(x0 + 1.0, 0.0, 63.0)
            cy0 = jnp.clip(y0, 0.0, 63.0)
            cy1 = jnp.clip(y0 + 1.0, 0.0, 63.0)
            corners = (
                (cy0, cx0, vy0 * vx0 * (1.0 - fx) * (1.0 - fy)),
                (cy1, cx0, vy1 * vx0 * (1.0 - fx) * fy),
                (cy0, cx1, vy0 * vx1 * fx * (1.0 - fy)),
                (cy1, cx1, vy1 * vx1 * fx * fy),
            )
            for ci, (cy, cx, wc) in enumerate(corners):
                pos = cy * 64.0 + cx
                idxf = (pos * float(NH) + (base + float(hd)))
                idx_ref[0, hd, p * 4 + ci, 0] = idxf.astype(jnp.int32)
                wt_ref[0, hd, p * 4 + ci, 0] = ap * wc


def _prep_call(query, W_off, b_off, W_attn, b_attn, rx, ry, wh):
    grid = (BS, _QT)
    kwargs = dict(
        grid=grid,
        in_specs=[
            pl.BlockSpec((1, 1, ED), lambda b, t: (b, 0, 0)),
            pl.BlockSpec((ED, NH * NP * 2), lambda b, t: (0, 0)),
            pl.BlockSpec((1, NH * NP * 2), lambda b, t: (0, 0)),
            pl.BlockSpec((ED, NH * NP), lambda b, t: (0, 0)),
            pl.BlockSpec((1, NH * NP), lambda b, t: (0, 0)),
            pl.BlockSpec((1, _QTR, 128), lambda b, t: (b, t, 0)),
            pl.BlockSpec((1, _QTR, 128), lambda b, t: (b, t, 0)),
            pl.BlockSpec((1, 2), lambda b, t: (0, 0)),
        ],
        out_specs=[
            pl.BlockSpec((1, NH, NSLOT, 1, _QTR, 128),
                         lambda b, t: (b, 0, 0, t, 0, 0)),
            pl.BlockSpec((1, NH, NSLOT, 1, _QTR, 128),
                         lambda b, t: (b, 0, 0, t, 0, 0)),
        ],
        out_shape=[
            jax.ShapeDtypeStruct((BS, NH, NSLOT, _QT, _QTR, 128), jnp.int32),
            jax.ShapeDtypeStruct((BS, NH, NSLOT, _QT, _QTR, 128), jnp.float32),
        ],
    )
    return pl.pallas_call(_prep_body, **kwargs)(
        query, W_off, b_off, W_attn, b_attn, rx, ry, wh)


def _sc_body(table_hbm, idx_hbm, wt_hbm, out_hbm, idx_v, wt_v, rows_v, out_v,
             sem_a, sem_b, sem2):
    sems = (sem_a, sem_b)
    wid = lax.axis_index("s") * NC + lax.axis_index("c")
    nch = NQ // CH

    def stage_iw(buf, ch):
        qb = ch * CH
        pltpu.async_copy(idx_hbm.at[wid, :, pl.ds(qb, CH)], idx_v.at[buf],
                         sem2)
        pltpu.async_copy(wt_hbm.at[wid, :, pl.ds(qb, CH)], wt_v.at[buf], sem2)

    def drain_iw(buf):
        pltpu.make_async_copy(idx_hbm.at[0, :, pl.ds(0, CH)], idx_v.at[buf],
                              sem2).wait()
        pltpu.make_async_copy(wt_hbm.at[0, :, pl.ds(0, CH)], wt_v.at[buf],
                              sem2).wait()

    def fire(buf):
        for j in range(NSLOT):
            pltpu.async_copy(table_hbm.at[idx_v.at[buf, j]],
                             rows_v.at[buf, j], sems[buf])

    def drain_rows(buf):
        for j in range(NSLOT):
            pltpu.make_async_copy(table_hbm.at[idx_v.at[buf, j]],
                                  rows_v.at[buf, j], sems[buf]).wait()

    def compute(buf, ch):
        qb = ch * CH

        def group_body(g, c2):
            qo = g * 16
            wvs = [wt_v[buf, j, pl.ds(qo, 16)] for j in range(NSLOT)]
            for qi in range(16):
                q = qo + qi
                acc0 = jnp.zeros((16,), jnp.float32)
                acc1 = jnp.zeros((16,), jnp.float32)
                for j in range(NSLOT):
                    w = wvs[j][qi]
                    acc0 = acc0 + w * rows_v[buf, j, q, pl.ds(0, 16)]
                    acc1 = acc1 + w * rows_v[buf, j, q, pl.ds(16, 16)]
                out_v[q, pl.ds(0, 16)] = acc0
                out_v[q, pl.ds(16, 16)] = acc1
            return c2

        lax.fori_loop(0, CH // 16, group_body, 0)
        pltpu.sync_copy(out_v, out_hbm.at[wid, pl.ds(qb, CH)])

    stage_iw(0, 0)
    drain_iw(0)
    fire(0)
    stage_iw(1, 1)

    def body(cc, carry):
        ch0 = cc * 2
        ch1 = ch0 + 1
        nxt0 = ch0 + 2
        nxt1 = ch1 + 2
        drain_iw(1)
        fire(1)
        drain_rows(0)
        compute(0, ch0)

        @pl.when(nxt0 < nch)
        def _():
            stage_iw(0, nxt0)
            drain_iw(0)
            fire(0)

        drain_rows(1)
        compute(1, ch1)

        @pl.when(nxt1 < nch)
        def _():
            stage_iw(1, nxt1)

        return carry

    lax.fori_loop(0, nch // 2, body, 0)


def _sc_call(table, idxs, wts):
    mesh = plsc.VectorSubcoreMesh(core_axis_name="c", subcore_axis_name="s")
    kern = functools.partial(
        pl.kernel, mesh=mesh,
        compiler_params=pltpu.CompilerParams(use_tc_tiling_on_sc=False),
        out_type=jax.ShapeDtypeStruct((BS * NH, NQ, HD), jnp.float32),
        scratch_types=[
            pltpu.VMEM((2, NSLOT, CH), jnp.int32),
            pltpu.VMEM((2, NSLOT, CH), jnp.float32),
            pltpu.VMEM((2, NSLOT, CH, HD), jnp.float32),
            pltpu.VMEM((CH, HD), jnp.float32),
            pltpu.SemaphoreType.DMA,
            pltpu.SemaphoreType.DMA,
            pltpu.SemaphoreType.DMA,
        ],
    )(_sc_body)
    return kern(table, idxs, wts)


def _post_body(o_ref, q_ref, g_ref, b_ref, y_ref):
    xs = [o_ref[0, hd] for hd in range(NH)]
    x = jnp.concatenate(xs, axis=-1)
    x = x + q_ref[0]
    mu = jnp.mean(x, axis=-1, keepdims=True)
    d = x - mu
    var = jnp.mean(d * d, axis=-1, keepdims=True)
    xn = d * lax.rsqrt(var + 1e-5)
    y_ref[0] = xn * g_ref[...] + b_ref[...]


def _post_call(out_bh, query, gamma, beta):
    T = NQ // _QT
    grid = (BS, _QT)
    return pl.pallas_call(
        _post_body,
        grid=grid,
        in_specs=[
            pl.BlockSpec((1, NH, T, HD), lambda b, t: (b, 0, t, 0)),
            pl.BlockSpec((1, 1, ED), lambda b, t: (b, 0, 0)),
            pl.BlockSpec((1, ED), lambda b, t: (0, 0)),
            pl.BlockSpec((1, ED), lambda b, t: (0, 0)),
        ],
        out_specs=pl.BlockSpec((1, T, ED), lambda b, t: (b, t, 0)),
        out_shape=jax.ShapeDtypeStruct((BS, NQ, ED), jnp.float32),
    )(out_bh, query, gamma, beta)


def kernel(query, single_feat, ref_2d, h, w, W_off, b_off, W_attn, b_attn,
           ln_gamma, ln_beta):
    rx = ref_2d[:, :, 0, 0].reshape(BS, NQ // 128, 128)
    ry = ref_2d[:, :, 0, 1].reshape(BS, NQ // 128, 128)
    wh = jnp.stack([jnp.asarray(w), jnp.asarray(h)]).astype(jnp.float32)
    wh = wh.reshape(1, 2)
    idx, wt = _prep_call(query, W_off, b_off.reshape(1, -1), W_attn,
                         b_attn.reshape(1, -1), rx, ry, wh)
    table = single_feat.reshape(BS * NQ * NH, HD)
    out_bh = _sc_call(table,
                      idx.reshape(BS * NH, NSLOT, NQ),
                      wt.reshape(BS * NH, NSLOT, NQ))
    y = _post_call(out_bh.reshape(BS, NH, NQ, HD), query,
                   ln_gamma.reshape(1, ED), ln_beta.reshape(1, ED))
    return y


# speedup vs baseline: 1.1979x; 1.0016x over previous
import functools

import jax
import jax.numpy as jnp
from jax import lax
from jax.experimental import pallas as pl
from jax.experimental.pallas import tpu as pltpu
from jax.experimental.pallas import tpu_sc as plsc

BS = 4
NQ = 4096
NH = 8
NP = 4
HD = 32
ED = 256
GS = 64
NSLOT = NP * 4

_QT = 4
_QTR = NQ // _QT // 128

NC, NS = 2, 16
CH = 64


def _prep_body(q_ref, wof_ref, bof_ref, wat_ref, bat_ref, rx_ref, ry_ref,
               wh_ref, idx_ref, wt_ref):
    b = pl.program_id(0)
    qv = q_ref[0]
    so = jnp.maximum(
        jnp.dot(qv, wof_ref[...], preferred_element_type=jnp.float32)
        + bof_ref[...], 0.0)
    aw = jnp.maximum(
        jnp.dot(qv, wat_ref[...], preferred_element_type=jnp.float32)
        + bat_ref[...], 0.0)
    sx = 64.0 / wh_ref[0:1, 0:1]
    sy = 64.0 / wh_ref[0:1, 1:2]
    rx = rx_ref[0]
    ry = ry_ref[0]
    base = (b * NQ * NH).astype(jnp.float32)
    for hd in range(NH):
        a = aw[:, hd * NP:(hd + 1) * NP]
        m = jnp.max(a, axis=-1, keepdims=True)
        e = jnp.exp(a - m)
        s = jnp.sum(e, axis=-1, keepdims=True)
        for p in range(NP):
            ap = e[:, p:p + 1] / s
            o = (hd * NP + p) * 2
            ox = so[:, o:o + 1] * sx
            oy = so[:, o + 1:o + 2] * sy
            x = rx * 64.0 + (ox - 0.5)
            y = ry * 64.0 + (oy - 0.5)
            x0 = jnp.floor(x)
            fx = x - x0
            y0 = jnp.floor(y)
            fy = y - y0
            vx0 = ((x0 >= 0.0) & (x0 <= 63.0)).astype(jnp.float32)
            vx1 = ((x0 >= -1.0) & (x0 <= 62.0)).astype(jnp.float32)
            vy0 = ((y0 >= 0.0) & (y0 <= 63.0)).astype(jnp.float32)
            vy1 = ((y0 >= -1.0) & (y0 <= 62.0)).astype(jnp.float32)
            cx0 = jnp.clip(x0, 0.0, 63.0)
            cx1 = jnp.clip(x0 + 1.0, 0.0, 63.0)
            cy0 = jnp.clip(y0, 0.0, 63.0)
            cy1 = jnp.clip(y0 + 1.0, 0.0, 63.0)
            corners = (
                (cy0, cx0, vy0 * vx0 * (1.0 - fx) * (1.0 - fy)),
                (cy1, cx0, vy1 * vx0 * (1.0 - fx) * fy),
                (cy0, cx1, vy0 * vx1 * fx * (1.0 - fy)),
                (cy1, cx1, vy1 * vx1 * fx * fy),
            )
            for ci, (cy, cx, wc) in enumerate(corners):
                pos = cy * 64.0 + cx
                idxf = (pos * float(NH) + (base + float(hd)))
                idx_ref[0, hd, p * 4 + ci, 0] = idxf.astype(jnp.int32)
                wt_ref[0, hd, p * 4 + ci, 0] = ap * wc


def _prep_call(query, W_off, b_off, W_attn, b_attn, rx, ry, wh):
    grid = (BS, _QT)
    kwargs = dict(
        grid=grid,
        in_specs=[
            pl.BlockSpec((1, 1, ED), lambda b, t: (b, 0, 0)),
            pl.BlockSpec((ED, NH * NP * 2), lambda b, t: (0, 0)),
            pl.BlockSpec((1, NH * NP * 2), lambda b, t: (0, 0)),
            pl.BlockSpec((ED, NH * NP), lambda b, t: (0, 0)),
            pl.BlockSpec((1, NH * NP), lambda b, t: (0, 0)),
            pl.BlockSpec((1, _QTR, 128), lambda b, t: (b, t, 0)),
            pl.BlockSpec((1, _QTR, 128), lambda b, t: (b, t, 0)),
            pl.BlockSpec((1, 2), lambda b, t: (0, 0)),
        ],
        out_specs=[
            pl.BlockSpec((1, NH, NSLOT, 1, _QTR, 128),
                         lambda b, t: (b, 0, 0, t, 0, 0)),
            pl.BlockSpec((1, NH, NSLOT, 1, _QTR, 128),
                         lambda b, t: (b, 0, 0, t, 0, 0)),
        ],
        out_shape=[
            jax.ShapeDtypeStruct((BS, NH, NSLOT, _QT, _QTR, 128), jnp.int32),
            jax.ShapeDtypeStruct((BS, NH, NSLOT, _QT, _QTR, 128), jnp.float32),
        ],
    )
    return pl.pallas_call(_prep_body, **kwargs)(
        query, W_off, b_off, W_attn, b_attn, rx, ry, wh)


def _sc_body(table_hbm, idx_hbm, wt_hbm, out_hbm, idx_v, wt_v, rows_v, out_v,
             sem_a, sem_b, sem2):
    sems = (sem_a, sem_b)
    wid = lax.axis_index("s") * NC + lax.axis_index("c")
    nch = NQ // CH

    def stage_iw(buf, ch):
        qb = ch * CH
        pltpu.async_copy(idx_hbm.at[wid, :, pl.ds(qb, CH)], idx_v.at[buf],
                         sem2)
        pltpu.async_copy(wt_hbm.at[wid, :, pl.ds(qb, CH)], wt_v.at[buf], sem2)

    def drain_iw(buf):
        pltpu.make_async_copy(idx_hbm.at[0, :, pl.ds(0, CH)], idx_v.at[buf],
                              sem2).wait()
        pltpu.make_async_copy(wt_hbm.at[0, :, pl.ds(0, CH)], wt_v.at[buf],
                              sem2).wait()

    def fire(buf):
        for j in range(NSLOT):
            pltpu.async_copy(table_hbm.at[idx_v.at[buf, j]],
                             rows_v.at[buf, j], sems[buf])

    def drain_rows(buf):
        for j in range(NSLOT):
            pltpu.make_async_copy(table_hbm.at[idx_v.at[buf, j]],
                                  rows_v.at[buf, j], sems[buf]).wait()

    def compute(buf, ch):
        qb = ch * CH

        @plsc.parallel_loop(0, CH // 16, 1, unroll=1)
        def group_body(g):
            qo = g * 16
            wvs = [wt_v[buf, j, pl.ds(qo, 16)] for j in range(NSLOT)]
            for qi in range(16):
                q = qo + qi
                acc0 = jnp.zeros((16,), jnp.float32)
                acc1 = jnp.zeros((16,), jnp.float32)
                for j in range(NSLOT):
                    w = wvs[j][qi]
                    acc0 = acc0 + w * rows_v[buf, j, q, pl.ds(0, 16)]
                    acc1 = acc1 + w * rows_v[buf, j, q, pl.ds(16, 16)]
                out_v[q, pl.ds(0, 16)] = acc0
                out_v[q, pl.ds(16, 16)] = acc1
        pltpu.sync_copy(out_v, out_hbm.at[wid, pl.ds(qb, CH)])

    stage_iw(0, 0)
    drain_iw(0)
    fire(0)
    stage_iw(1, 1)

    def body(cc, carry):
        ch0 = cc * 2
        ch1 = ch0 + 1
        nxt0 = ch0 + 2
        nxt1 = ch1 + 2
        drain_iw(1)
        fire(1)
        drain_rows(0)
        compute(0, ch0)

        @pl.when(nxt0 < nch)
        def _():
            stage_iw(0, nxt0)
            drain_iw(0)
            fire(0)

        drain_rows(1)
        compute(1, ch1)

        @pl.when(nxt1 < nch)
        def _():
            stage_iw(1, nxt1)

        return carry

    lax.fori_loop(0, nch // 2, body, 0)


def _sc_call(table, idxs, wts):
    mesh = plsc.VectorSubcoreMesh(core_axis_name="c", subcore_axis_name="s")
    kern = functools.partial(
        pl.kernel, mesh=mesh,
        compiler_params=pltpu.CompilerParams(use_tc_tiling_on_sc=False),
        out_type=jax.ShapeDtypeStruct((BS * NH, NQ, HD), jnp.float32),
        scratch_types=[
            pltpu.VMEM((2, NSLOT, CH), jnp.int32),
            pltpu.VMEM((2, NSLOT, CH), jnp.float32),
            pltpu.VMEM((2, NSLOT, CH, HD), jnp.float32),
            pltpu.VMEM((CH, HD), jnp.float32),
            pltpu.SemaphoreType.DMA,
            pltpu.SemaphoreType.DMA,
            pltpu.SemaphoreType.DMA,
        ],
    )(_sc_body)
    return kern(table, idxs, wts)


def _post_body(o_ref, q_ref, g_ref, b_ref, y_ref):
    xs = [o_ref[0, hd] for hd in range(NH)]
    x = jnp.concatenate(xs, axis=-1)
    x = x + q_ref[0]
    mu = jnp.mean(x, axis=-1, keepdims=True)
    d = x - mu
    var = jnp.mean(d * d, axis=-1, keepdims=True)
    xn = d * lax.rsqrt(var + 1e-5)
    y_ref[0] = xn * g_ref[...] + b_ref[...]


def _post_call(out_bh, query, gamma, beta):
    T = NQ // _QT
    grid = (BS, _QT)
    return pl.pallas_call(
        _post_body,
        grid=grid,
        in_specs=[
            pl.BlockSpec((1, NH, T, HD), lambda b, t: (b, 0, t, 0)),
            pl.BlockSpec((1, 1, ED), lambda b, t: (b, 0, 0)),
            pl.BlockSpec((1, ED), lambda b, t: (0, 0)),
            pl.BlockSpec((1, ED), lambda b, t: (0, 0)),
        ],
        out_specs=pl.BlockSpec((1, T, ED), lambda b, t: (b, t, 0)),
        out_shape=jax.ShapeDtypeStruct((BS, NQ, ED), jnp.float32),
    )(out_bh, query, gamma, beta)


def kernel(query, single_feat, ref_2d, h, w, W_off, b_off, W_attn, b_attn,
           ln_gamma, ln_beta):
    rx = ref_2d[:, :, 0, 0].reshape(BS, NQ // 128, 128)
    ry = ref_2d[:, :, 0, 1].reshape(BS, NQ // 128, 128)
    wh = jnp.stack([jnp.asarray(w), jnp.asarray(h)]).astype(jnp.float32)
    wh = wh.reshape(1, 2)
    idx, wt = _prep_call(query, W_off, b_off.reshape(1, -1), W_attn,
                         b_attn.reshape(1, -1), rx, ry, wh)
    table = single_feat.reshape(BS * NQ * NH, HD)
    out_bh = _sc_call(table,
                      idx.reshape(BS * NH, NSLOT, NQ),
                      wt.reshape(BS * NH, NSLOT, NQ))
    y = _post_call(out_bh.reshape(BS, NH, NQ, HD), query,
                   ln_gamma.reshape(1, ED), ln_beta.reshape(1, ED))
    return y
